# baseline (speedup 1.0000x reference)
import functools

import jax
import jax.numpy as jnp
from jax import lax
from jax.experimental import pallas as pl
from jax.experimental.pallas import tpu as pltpu
from jax.experimental.pallas import tpu_sc as plsc

VOCAB = 30522
D = 768
BATCH = 128
SEQ = 512
B = BATCH * SEQ

NC = 2
NS = 16
NW = NC * NS
LANES = 16

S_PER_W = SEQ // NW
ROWS_PER_W = B // NW
CHUNK = 64
CHUNKS_PER_S = BATCH // CHUNK
N_CHUNKS = ROWS_PER_W // CHUNK
VREGS_PER_ROW = D // LANES


def _sc_body(seq_hbm, table_hbm, pos_hbm, out_hbm, idx_all, pos_v, rows_v, sem):
    wid = lax.axis_index("s") * NC + lax.axis_index("c")
    pltpu.sync_copy(seq_hbm.at[pl.ds(wid * ROWS_PER_W, ROWS_PER_W)], idx_all)
    pltpu.sync_copy(pos_hbm.at[pl.ds(wid * S_PER_W, S_PER_W)], pos_v)

    def chunk_step(c, carry):
        s_loc = c // CHUNKS_PER_S
        b0 = (c % CHUNKS_PER_S) * CHUNK
        idx_slice = idx_all.at[pl.ds(c * CHUNK, CHUNK)]
        pltpu.async_copy(table_hbm.at[idx_slice], rows_v, sem).wait()

        for j in range(VREGS_PER_ROW):
            p = pos_v[s_loc, pl.ds(j * LANES, LANES)]

            def row_step(i, carry2):
                g = rows_v[i, pl.ds(j * LANES, LANES)]
                rows_v[i, pl.ds(j * LANES, LANES)] = (g + p) * 2.0
                return carry2

            lax.fori_loop(0, CHUNK, row_step, 0, unroll=False)

        s_glob = wid * S_PER_W + s_loc
        pltpu.sync_copy(rows_v, out_hbm.at[pl.ds(b0, CHUNK), s_glob])
        return carry

    lax.fori_loop(0, N_CHUNKS, chunk_step, 0, unroll=False)


@jax.jit
def _embed(seq_t, content_table, pos_pe):
    mesh = plsc.VectorSubcoreMesh(core_axis_name="c", subcore_axis_name="s")
    k = functools.partial(
        pl.kernel,
        mesh=mesh,
        out_type=jax.ShapeDtypeStruct((BATCH, SEQ, D), jnp.float32),
        scratch_types=[
            pltpu.VMEM((ROWS_PER_W,), jnp.int32),
            pltpu.VMEM((S_PER_W, D), jnp.float32),
            pltpu.VMEM((CHUNK, D), jnp.float32),
            pltpu.SemaphoreType.DMA,
        ],
    )(_sc_body)
    return k(seq_t, content_table, pos_pe)


def kernel(sequence, content_table, pos_pe):
    seq_t = jnp.swapaxes(sequence, 0, 1).reshape(B)
    return _embed(seq_t, content_table, pos_pe)


# speedup vs baseline: 5.1099x; 5.1099x over previous
import functools

import jax
import jax.numpy as jnp
from jax import lax
from jax.experimental import pallas as pl
from jax.experimental.pallas import tpu as pltpu
from jax.experimental.pallas import tpu_sc as plsc

VOCAB = 30522
D = 768
BATCH = 128
SEQ = 512

NC = 2
NS = 16
NW = NC * NS
LANES = 16

BG = 2
SG = NW // BG
B_PER_W = BATCH // BG
S_PER_W = SEQ // SG
NBUF = 4
VREGS_PER_ROW = D // LANES


def _sc_body(seq_hbm, table_hbm, pos_hbm, out_hbm,
             idx_all, pos_v, g0, g1, g2, g3,
             gs0, gs1, gs2, gs3, ws0, ws1, ws2, ws3):
    wid = lax.axis_index("s") * NC + lax.axis_index("c")
    bg = wid // SG
    sg = lax.rem(wid, SG)
    b0 = bg * B_PER_W
    s0 = sg * S_PER_W

    bufs = (g0, g1, g2, g3)
    gsems = (gs0, gs1, gs2, gs3)
    wsems = (ws0, ws1, ws2, ws3)

    pltpu.sync_copy(seq_hbm.at[wid], idx_all)
    pltpu.sync_copy(pos_hbm.at[pl.ds(s0, S_PER_W)], pos_v)

    def start_gather(c, k):
        pltpu.async_copy(table_hbm.at[idx_all.at[c]], bufs[k], gsems[k])

    for k in range(NBUF - 1):
        start_gather(k, k)

    def step(t, carry):
        for k in range(NBUF):
            c = t * NBUF + k
            pltpu.make_async_copy(table_hbm.at[idx_all.at[c]], bufs[k],
                                  gsems[k]).wait()
            buf = bufs[k]

            def row_step(i, carry2):
                for j in range(VREGS_PER_ROW):
                    g = buf[i, pl.ds(j * LANES, LANES)]
                    p = pos_v[i, pl.ds(j * LANES, LANES)]
                    buf[i, pl.ds(j * LANES, LANES)] = (g + p) * 2.0
                return carry2

            lax.fori_loop(0, S_PER_W, row_step, 0, unroll=False)

            dst = out_hbm.at[b0 + c, pl.ds(s0, S_PER_W)]
            pltpu.async_copy(buf, dst, wsems[k])

            kn = (k + NBUF - 1) % NBUF
            cn = c + NBUF - 1

            @pl.when(jnp.logical_and(c >= 1, cn < B_PER_W))
            def _wait_prev_write():
                prev = out_hbm.at[b0 + c - 1, pl.ds(s0, S_PER_W)]
                pltpu.make_async_copy(bufs[kn], prev, wsems[kn]).wait()

            @pl.when(cn < B_PER_W)
            def _prefetch():
                start_gather(cn, kn)
        return carry

    lax.fori_loop(0, B_PER_W // NBUF, step, 0, unroll=False)

    for k in range(NBUF):
        c = B_PER_W - NBUF + k
        dst = out_hbm.at[b0 + c, pl.ds(s0, S_PER_W)]
        pltpu.make_async_copy(bufs[k], dst, wsems[k]).wait()


@jax.jit
def _embed(seq_blocks, content_table, pos_pe):
    mesh = plsc.VectorSubcoreMesh(core_axis_name="c", subcore_axis_name="s")
    k = functools.partial(
        pl.kernel,
        mesh=mesh,
        out_type=jax.ShapeDtypeStruct((BATCH, SEQ, D), jnp.float32),
        scratch_types=[
            pltpu.VMEM((B_PER_W, S_PER_W), jnp.int32),
            pltpu.VMEM((S_PER_W, D), jnp.float32),
        ] + [pltpu.VMEM((S_PER_W, D), jnp.float32)] * NBUF
          + [pltpu.SemaphoreType.DMA] * (2 * NBUF),
    )(_sc_body)
    return k(seq_blocks, content_table, pos_pe)


def kernel(sequence, content_table, pos_pe):
    seq_blocks = (sequence.reshape(BG, B_PER_W, SG, S_PER_W)
                  .transpose(0, 2, 1, 3).reshape(NW, B_PER_W, S_PER_W))
    return _embed(seq_blocks, content_table, pos_pe)


# speedup vs baseline: 5.1404x; 1.0060x over previous
import functools

import jax
import jax.numpy as jnp
from jax import lax
from jax.experimental import pallas as pl
from jax.experimental.pallas import tpu as pltpu
from jax.experimental.pallas import tpu_sc as plsc

VOCAB = 30522
D = 768
BATCH = 128
SEQ = 512
B = BATCH * SEQ

NC = 2
NS = 16
NW = NC * NS
LANES = 16

BG = 2
SG = NW // BG
B_PER_W = BATCH // BG
S_PER_W = SEQ // SG
NBUF = 4
VREGS_PER_ROW = D // LANES


def _sc_body(seq_hbm, table_hbm, pos_hbm, out_hbm,
             idx_bufs, pos_v, g0, g1, g2, g3,
             gs0, gs1, gs2, gs3, ws0, ws1, ws2, ws3,
             is0, is1, is2, is3):
    wid = lax.axis_index("s") * NC + lax.axis_index("c")
    bg = wid // SG
    sg = lax.rem(wid, SG)
    b0 = bg * B_PER_W
    s0 = sg * S_PER_W

    bufs = (g0, g1, g2, g3)
    gsems = (gs0, gs1, gs2, gs3)
    wsems = (ws0, ws1, ws2, ws3)
    isems = (is0, is1, is2, is3)

    pltpu.sync_copy(pos_hbm.at[pl.ds(s0, S_PER_W)], pos_v)

    def idx_src(c):
        return seq_hbm.at[pl.ds((b0 + c) * SEQ + s0, S_PER_W)]

    def start_idx(c, k):
        pltpu.async_copy(idx_src(c), idx_bufs.at[k], isems[k])

    def wait_idx(c, k):
        pltpu.make_async_copy(idx_src(c), idx_bufs.at[k], isems[k]).wait()

    def start_gather(c, k):
        pltpu.async_copy(table_hbm.at[idx_bufs.at[k]], bufs[k], gsems[k])

    def wait_gather(c, k):
        pltpu.make_async_copy(table_hbm.at[idx_bufs.at[k]], bufs[k],
                              gsems[k]).wait()

    for k in range(NBUF):
        start_idx(k, k)
    for k in range(NBUF - 1):
        wait_idx(k, k)
        start_gather(k, k)

    def step(t, carry):
        for k in range(NBUF):
            c = t * NBUF + k
            wait_gather(c, k)
            buf = bufs[k]

            def row_step(i, carry2):
                for j in range(VREGS_PER_ROW):
                    g = buf[i, pl.ds(j * LANES, LANES)]
                    p = pos_v[i, pl.ds(j * LANES, LANES)]
                    buf[i, pl.ds(j * LANES, LANES)] = (g + p) * 2.0
                return carry2

            lax.fori_loop(0, S_PER_W, row_step, 0, unroll=False)

            dst = out_hbm.at[b0 + c, pl.ds(s0, S_PER_W)]
            pltpu.async_copy(buf, dst, wsems[k])

            @pl.when(c + NBUF < B_PER_W)
            def _prefetch_idx():
                start_idx(c + NBUF, k)

            kn = (k + NBUF - 1) % NBUF
            cn = c + NBUF - 1

            @pl.when(jnp.logical_and(c >= 1, cn < B_PER_W))
            def _wait_prev_write():
                prev = out_hbm.at[b0 + c - 1, pl.ds(s0, S_PER_W)]
                pltpu.make_async_copy(bufs[kn], prev, wsems[kn]).wait()

            @pl.when(cn < B_PER_W)
            def _start_gather():
                wait_idx(cn, kn)
                start_gather(cn, kn)
        return carry

    lax.fori_loop(0, B_PER_W // NBUF, step, 0, unroll=False)

    for k in range(NBUF):
        c = B_PER_W - NBUF + k
        dst = out_hbm.at[b0 + c, pl.ds(s0, S_PER_W)]
        pltpu.make_async_copy(bufs[k], dst, wsems[k]).wait()


@jax.jit
def _embed(seq_flat, content_table, pos_pe):
    mesh = plsc.VectorSubcoreMesh(core_axis_name="c", subcore_axis_name="s")
    k = functools.partial(
        pl.kernel,
        mesh=mesh,
        out_type=jax.ShapeDtypeStruct((BATCH, SEQ, D), jnp.float32),
        scratch_types=[
            pltpu.VMEM((NBUF, S_PER_W), jnp.int32),
            pltpu.VMEM((S_PER_W, D), jnp.float32),
        ] + [pltpu.VMEM((S_PER_W, D), jnp.float32)] * NBUF
          + [pltpu.SemaphoreType.DMA] * (3 * NBUF),
    )(_sc_body)
    return k(seq_flat, content_table, pos_pe)


def kernel(sequence, content_table, pos_pe):
    return _embed(sequence.reshape(B), content_table, pos_pe)
